# baseline (speedup 1.0000x reference)
import functools

import jax
import jax.numpy as jnp
from jax.experimental import pallas as pl
from jax.experimental.pallas import tpu as pltpu

_PAD = 3
_KSIZE = 3
_BN_EPS = 1e-5
_GUARD = 3


def _conv_stats_kernel(x_ref, wv_ref, wh_ref, bv_ref, bh_ref, m_ref,
                       feat_ref, stats_ref, *, v_starts, h_starts, L):
    xv = jnp.concatenate([x_ref[0, :, s:s + L] for s in v_starts], axis=0)
    xh = jnp.concatenate([x_ref[0, :, s:s + L] for s in h_starts], axis=0)
    yv = jnp.dot(wv_ref[...], xv, preferred_element_type=jnp.float32)
    yh = jnp.dot(wh_ref[...], xh, preferred_element_type=jnp.float32)
    ym = (jnp.concatenate([yv + bv_ref[...], yh + bh_ref[...]], axis=0)
          * m_ref[...])
    feat_ref[0] = ym.astype(feat_ref.dtype)
    ssum = jnp.sum(ym, axis=1, keepdims=True)
    ssq = jnp.sum(ym * ym, axis=1, keepdims=True)
    stats_ref[0] = jnp.concatenate([ssum, ssq], axis=1)


def _bn_relu_kernel(feat_ref, sc_ref, sh_ref, o_ref, *, W):
    f = feat_ref[0][:, :, _PAD:_PAD + W].astype(jnp.float32)
    sc = sc_ref[...][:, :, :W]
    sh = sh_ref[...][:, :, :W]
    o_ref[0] = jnp.maximum(f * sc + sh, 0.0)


def _pack_taps(ws, dils, Cin):
    cg = sum(w.shape[0] for w in ws)
    wf = jnp.zeros((cg, 7 * Cin), jnp.float32)
    co = 0
    for w, dil in zip(ws, dils):
        c = w.shape[0]
        w2 = w.reshape(c, Cin, _KSIZE).astype(jnp.float32)
        for t in range(_KSIZE):
            s = 3 + (t - 1) * dil
            wf = wf.at[co:co + c, s * Cin:(s + 1) * Cin].set(w2[:, :, t])
        co += c
    return wf


def kernel(x, w_first, w_second, w_third, w_first2, w_second2, w_third2,
           b_first, b_second, b_third, b_first2, b_second2, b_third2,
           gamma, beta):
    N, Cin, H, W = x.shape
    Cout = gamma.shape[0]

    extra = 0
    for e in range(33):
        if (H * (W + 2 * _PAD + e)) % 128 == 0:
            extra = e
            break
    Wp = W + 2 * _PAD + extra
    Hp = H + 2 * _PAD
    L = H * Wp
    Lp = Hp * Wp + 2 * _GUARD

    xp = jnp.pad(x.astype(jnp.bfloat16),
                 ((0, 0), (0, 0), (_PAD, _PAD), (_PAD, _PAD + extra)))
    xf = jnp.pad(xp.reshape(N, Cin, Hp * Wp), ((0, 0), (0, 0), (_GUARD, _GUARD)))

    v_starts = tuple(_GUARD + k * Wp for k in range(7))
    h_starts = tuple(_GUARD + _PAD * Wp + (k - 3) for k in range(7))

    wv = _pack_taps([w_first, w_second, w_third], (1, 2, 3), Cin)
    wh = _pack_taps([w_first2, w_second2, w_third2], (1, 2, 3), Cin)
    wv = wv.astype(jnp.bfloat16)
    wh = wh.astype(jnp.bfloat16)
    cv = wv.shape[0]
    bv = jnp.concatenate([b_first, b_second, b_third]).reshape(cv, 1)
    bh = jnp.concatenate([b_first2, b_second2, b_third2]).reshape(Cout - cv, 1)
    bv = bv.astype(jnp.float32)
    bh = bh.astype(jnp.float32)

    col = jnp.arange(L, dtype=jnp.int32) % Wp
    mask = ((col >= _PAD) & (col < _PAD + W)).astype(jnp.float32).reshape(1, L)

    Kv = wv.shape[1]
    Kh = wh.shape[1]
    ch = Cout - cv

    conv_fn = functools.partial(_conv_stats_kernel, v_starts=v_starts,
                                h_starts=h_starts, L=L)
    feat, stats = pl.pallas_call(
        conv_fn,
        out_shape=(jax.ShapeDtypeStruct((N, Cout, L), jnp.bfloat16),
                   jax.ShapeDtypeStruct((N, Cout, 2), jnp.float32)),
        grid=(N,),
        in_specs=[pl.BlockSpec((1, Cin, Lp), lambda n: (n, 0, 0)),
                  pl.BlockSpec((cv, Kv), lambda n: (0, 0)),
                  pl.BlockSpec((ch, Kh), lambda n: (0, 0)),
                  pl.BlockSpec((cv, 1), lambda n: (0, 0)),
                  pl.BlockSpec((ch, 1), lambda n: (0, 0)),
                  pl.BlockSpec((1, L), lambda n: (0, 0))],
        out_specs=(pl.BlockSpec((1, Cout, L), lambda n: (n, 0, 0)),
                   pl.BlockSpec((1, Cout, 2), lambda n: (n, 0, 0))),
        compiler_params=pltpu.CompilerParams(dimension_semantics=("parallel",)),
    )(xf, wv, wh, bv, bh, mask)

    tot = jnp.sum(stats, axis=0)
    inv_count = 1.0 / float(N * H * W)
    mean = tot[:, 0] * inv_count
    var = tot[:, 1] * inv_count - mean * mean
    scale = gamma.astype(jnp.float32) * jax.lax.rsqrt(var + jnp.float32(_BN_EPS))
    shift = beta.astype(jnp.float32) - mean * scale
    scale3 = jnp.broadcast_to(scale[:, None, None], (Cout, 1, Wp))
    shift3 = jnp.broadcast_to(shift[:, None, None], (Cout, 1, Wp))

    bn_fn = functools.partial(_bn_relu_kernel, W=W)
    out = pl.pallas_call(
        bn_fn,
        out_shape=jax.ShapeDtypeStruct((N, Cout, H, W), jnp.float32),
        grid=(N,),
        in_specs=[pl.BlockSpec((1, Cout, H, Wp), lambda n: (n, 0, 0, 0)),
                  pl.BlockSpec((Cout, 1, Wp), lambda n: (0, 0, 0)),
                  pl.BlockSpec((Cout, 1, Wp), lambda n: (0, 0, 0))],
        out_specs=pl.BlockSpec((1, Cout, H, W), lambda n: (n, 0, 0, 0)),
        compiler_params=pltpu.CompilerParams(dimension_semantics=("parallel",)),
    )(feat.reshape(N, Cout, H, Wp), scale3, shift3)
    return out


# speedup vs baseline: 2.2440x; 2.2440x over previous
import functools

import jax
import jax.numpy as jnp
from jax.experimental import pallas as pl
from jax.experimental.pallas import tpu as pltpu

_PAD = 3
_KSIZE = 3
_BN_EPS = 1e-5


def _conv_stats_kernel(xq_ref, wv_ref, wh_ref, b_ref, feat_ref, stats_ref, *,
                       offs_v, offs_h, H, W, WG):
    M = H * WG
    xv = jnp.concatenate([xq_ref[0, o:o + M, :] for o in offs_v], axis=1)
    xh = jnp.concatenate([xq_ref[0, o:o + M, :] for o in offs_h], axis=1)
    yv = jnp.dot(xv, wv_ref[...], preferred_element_type=jnp.float32)
    yh = jnp.dot(xh, wh_ref[...], preferred_element_type=jnp.float32)
    ym = jnp.concatenate([yv, yh], axis=1) + b_ref[...]
    feat_ref[0] = ym.astype(feat_ref.dtype)
    col = jax.lax.broadcasted_iota(jnp.int32, (H, WG, 1), 1)
    valid = (col >= _PAD) & (col < _PAD + W)
    ym3 = ym.reshape(H, WG, -1)
    ymm = jnp.where(valid, ym3, 0.0)
    ssum = jnp.sum(ymm, axis=(0, 1)).reshape(1, -1)
    ssq = jnp.sum(ymm * ymm, axis=(0, 1)).reshape(1, -1)
    stats_ref[0] = jnp.concatenate([ssum, ssq], axis=0)


def _bn_relu_kernel(feat_ref, sc_ref, sh_ref, o_ref, *, H, W, WG):
    z = jnp.maximum(feat_ref[0].astype(jnp.float32) * sc_ref[...] + sh_ref[...],
                    0.0)
    o_ref[0] = z.reshape(H, WG, -1)[:, _PAD:_PAD + W, :]


def _pack_taps(ws, dils, Cin):
    cg = sum(w.shape[0] for w in ws)
    wf = jnp.zeros((7 * Cin, cg), jnp.float32)
    co = 0
    for w, dil in zip(ws, dils):
        c = w.shape[0]
        w2 = w.reshape(c, Cin, _KSIZE).astype(jnp.float32)
        for t in range(_KSIZE):
            s = 3 + (t - 1) * dil
            wf = wf.at[s * Cin:(s + 1) * Cin, co:co + c].set(w2[:, :, t].T)
        co += c
    return wf


def kernel(x, w_first, w_second, w_third, w_first2, w_second2, w_third2,
           b_first, b_second, b_third, b_first2, b_second2, b_third2,
           gamma, beta):
    N, Cin, H, W = x.shape
    Cout = gamma.shape[0]

    WG = W + 2 * _PAD + 10
    HP = H + 2 * _PAD + 1
    M = H * WG
    R = HP * WG

    xt = jnp.transpose(x, (0, 2, 3, 1)).astype(jnp.bfloat16)
    xq = jnp.pad(xt, ((0, 0), (_PAD, _PAD + 1), (_PAD, WG - W - _PAD), (0, 0)))
    xq = xq.reshape(N, R, Cin)

    offs_v = tuple((_PAD + dh) * WG for dh in range(-3, 4))
    offs_h = tuple(_PAD * WG + dw for dw in range(-3, 4))

    wv = _pack_taps([w_first, w_second, w_third], (1, 2, 3), Cin).astype(jnp.bfloat16)
    wh = _pack_taps([w_first2, w_second2, w_third2], (1, 2, 3), Cin).astype(jnp.bfloat16)
    bias = jnp.concatenate([b_first, b_second, b_third,
                            b_first2, b_second2, b_third2])
    bias2 = bias.reshape(1, Cout).astype(jnp.float32)

    conv_fn = functools.partial(_conv_stats_kernel, offs_v=offs_v,
                                offs_h=offs_h, H=H, W=W, WG=WG)
    feat, stats = pl.pallas_call(
        conv_fn,
        out_shape=(jax.ShapeDtypeStruct((N, M, Cout), jnp.bfloat16),
                   jax.ShapeDtypeStruct((N, 2, Cout), jnp.float32)),
        grid=(N,),
        in_specs=[pl.BlockSpec((1, R, Cin), lambda n: (n, 0, 0)),
                  pl.BlockSpec(wv.shape, lambda n: (0, 0)),
                  pl.BlockSpec(wh.shape, lambda n: (0, 0)),
                  pl.BlockSpec((1, Cout), lambda n: (0, 0))],
        out_specs=(pl.BlockSpec((1, M, Cout), lambda n: (n, 0, 0)),
                   pl.BlockSpec((1, 2, Cout), lambda n: (n, 0, 0))),
        compiler_params=pltpu.CompilerParams(dimension_semantics=("parallel",)),
    )(xq, wv, wh, bias2)

    tot = jnp.sum(stats, axis=0)
    inv_count = 1.0 / float(N * H * W)
    mean = tot[0] * inv_count
    var = tot[1] * inv_count - mean * mean
    scale = gamma.astype(jnp.float32) * jax.lax.rsqrt(var + jnp.float32(_BN_EPS))
    shift = beta.astype(jnp.float32) - mean * scale
    scale2 = scale.reshape(1, Cout)
    shift2 = shift.reshape(1, Cout)

    bn_fn = functools.partial(_bn_relu_kernel, H=H, W=W, WG=WG)
    out = pl.pallas_call(
        bn_fn,
        out_shape=jax.ShapeDtypeStruct((N, H, W, Cout), jnp.float32),
        grid=(N,),
        in_specs=[pl.BlockSpec((1, M, Cout), lambda n: (n, 0, 0)),
                  pl.BlockSpec((1, Cout), lambda n: (0, 0)),
                  pl.BlockSpec((1, Cout), lambda n: (0, 0))],
        out_specs=pl.BlockSpec((1, H, W, Cout), lambda n: (n, 0, 0, 0)),
        compiler_params=pltpu.CompilerParams(dimension_semantics=("parallel",)),
    )(feat, scale2, shift2)
    return jnp.transpose(out, (0, 3, 1, 2))


# speedup vs baseline: 2.5696x; 1.1451x over previous
import functools

import jax
import jax.numpy as jnp
from jax.experimental import pallas as pl
from jax.experimental.pallas import tpu as pltpu

_PAD = 3
_KSIZE = 3
_BN_EPS = 1e-5


def _conv_stats_kernel(x_ref, wv_ref, wh_ref, b_ref, feat_ref, stats_ref,
                       scr_ref, *, offs_v, offs_h, H, W, WG):
    M = H * WG
    Cin = x_ref.shape[3]
    xr = x_ref[0].astype(jnp.bfloat16)
    zl = jnp.zeros((H, _PAD, Cin), jnp.bfloat16)
    zr = jnp.zeros((H, WG - W - _PAD, Cin), jnp.bfloat16)
    xrow = jnp.concatenate([zl, xr, zr], axis=1).reshape(M, Cin)
    top = _PAD * WG
    scr_ref[0:top, :] = jnp.zeros((top, Cin), jnp.bfloat16)
    scr_ref[top:top + M, :] = xrow
    scr_ref[top + M:, :] = jnp.zeros((scr_ref.shape[0] - top - M, Cin),
                                     jnp.bfloat16)
    xv = jnp.concatenate([scr_ref[o:o + M, :] for o in offs_v], axis=1)
    xh = jnp.concatenate([scr_ref[o:o + M, :] for o in offs_h], axis=1)
    yv = jnp.dot(xv, wv_ref[...], preferred_element_type=jnp.float32)
    yh = jnp.dot(xh, wh_ref[...], preferred_element_type=jnp.float32)
    ym = jnp.concatenate([yv, yh], axis=1)
    ys = ym.reshape(H, WG, -1)[:, _PAD:_PAD + W, :] + b_ref[...].reshape(1, 1, -1)
    feat_ref[0] = ys.reshape(H * W, -1).astype(feat_ref.dtype)
    ssum = jnp.sum(ys, axis=(0, 1)).reshape(1, -1)
    ssq = jnp.sum(ys * ys, axis=(0, 1)).reshape(1, -1)
    stats_ref[0] = jnp.concatenate([ssum, ssq], axis=0)


def _bn_relu_kernel(feat_ref, sc_ref, sh_ref, o_ref, *, H, W):
    z = jnp.maximum(feat_ref[0].astype(jnp.float32) * sc_ref[...] + sh_ref[...],
                    0.0)
    o_ref[0] = z.reshape(H, W, -1)


def _pack_taps(ws, dils, Cin):
    cg = sum(w.shape[0] for w in ws)
    wf = jnp.zeros((7 * Cin, cg), jnp.float32)
    co = 0
    for w, dil in zip(ws, dils):
        c = w.shape[0]
        w2 = w.reshape(c, Cin, _KSIZE).astype(jnp.float32)
        for t in range(_KSIZE):
            s = 3 + (t - 1) * dil
            wf = wf.at[s * Cin:(s + 1) * Cin, co:co + c].set(w2[:, :, t].T)
        co += c
    return wf


def kernel(x, w_first, w_second, w_third, w_first2, w_second2, w_third2,
           b_first, b_second, b_third, b_first2, b_second2, b_third2,
           gamma, beta):
    N, Cin, H, W = x.shape
    Cout = gamma.shape[0]

    WG = W + 2 * _PAD + 10
    HP = H + 2 * _PAD + 1
    M = H * WG
    R = HP * WG

    xt = jnp.transpose(x, (0, 2, 3, 1))

    offs_v = tuple((_PAD + dh) * WG for dh in range(-3, 4))
    offs_h = tuple(_PAD * WG + dw for dw in range(-3, 4))

    wv = _pack_taps([w_first, w_second, w_third], (1, 2, 3), Cin).astype(jnp.bfloat16)
    wh = _pack_taps([w_first2, w_second2, w_third2], (1, 2, 3), Cin).astype(jnp.bfloat16)
    bias = jnp.concatenate([b_first, b_second, b_third,
                            b_first2, b_second2, b_third2])
    bias2 = bias.reshape(1, Cout).astype(jnp.float32)

    conv_fn = functools.partial(_conv_stats_kernel, offs_v=offs_v,
                                offs_h=offs_h, H=H, W=W, WG=WG)
    feat, stats = pl.pallas_call(
        conv_fn,
        out_shape=(jax.ShapeDtypeStruct((N, H * W, Cout), jnp.bfloat16),
                   jax.ShapeDtypeStruct((N, 2, Cout), jnp.float32)),
        grid=(N,),
        in_specs=[pl.BlockSpec((1, H, W, Cin), lambda n: (n, 0, 0, 0)),
                  pl.BlockSpec(wv.shape, lambda n: (0, 0)),
                  pl.BlockSpec(wh.shape, lambda n: (0, 0)),
                  pl.BlockSpec((1, Cout), lambda n: (0, 0))],
        out_specs=(pl.BlockSpec((1, H * W, Cout), lambda n: (n, 0, 0)),
                   pl.BlockSpec((1, 2, Cout), lambda n: (n, 0, 0))),
        scratch_shapes=[pltpu.VMEM((R, Cin), jnp.bfloat16)],
        compiler_params=pltpu.CompilerParams(dimension_semantics=("parallel",)),
    )(xt, wv, wh, bias2)

    tot = jnp.sum(stats, axis=0)
    inv_count = 1.0 / float(N * H * W)
    mean = tot[0] * inv_count
    var = tot[1] * inv_count - mean * mean
    scale = gamma.astype(jnp.float32) * jax.lax.rsqrt(var + jnp.float32(_BN_EPS))
    shift = beta.astype(jnp.float32) - mean * scale
    scale2 = scale.reshape(1, Cout)
    shift2 = shift.reshape(1, Cout)

    bn_fn = functools.partial(_bn_relu_kernel, H=H, W=W)
    out = pl.pallas_call(
        bn_fn,
        out_shape=jax.ShapeDtypeStruct((N, H, W, Cout), jnp.float32),
        grid=(N,),
        in_specs=[pl.BlockSpec((1, H * W, Cout), lambda n: (n, 0, 0)),
                  pl.BlockSpec((1, Cout), lambda n: (0, 0)),
                  pl.BlockSpec((1, Cout), lambda n: (0, 0))],
        out_specs=pl.BlockSpec((1, H, W, Cout), lambda n: (n, 0, 0, 0)),
        compiler_params=pltpu.CompilerParams(dimension_semantics=("parallel",)),
    )(feat, scale2, shift2)
    return jnp.transpose(out, (0, 3, 1, 2))


# speedup vs baseline: 2.7040x; 1.0523x over previous
import functools

import jax
import jax.numpy as jnp
from jax.experimental import pallas as pl
from jax.experimental.pallas import tpu as pltpu

_PAD = 3
_KSIZE = 3
_BN_EPS = 1e-5


def _conv_stats_kernel(x_ref, wv_ref, wh_ref, b_ref, feat_ref, stats_ref,
                       scr_ref, *, offs_v, offs_h, H, W, WG):
    M = H * WG
    Cin = x_ref.shape[3]
    xr = x_ref[0].astype(jnp.bfloat16)
    zl = jnp.zeros((H, _PAD, Cin), jnp.bfloat16)
    zr = jnp.zeros((H, WG - W - _PAD, Cin), jnp.bfloat16)
    xrow = jnp.concatenate([zl, xr, zr], axis=1).reshape(M, Cin)
    top = _PAD * WG
    scr_ref[0:top, :] = jnp.zeros((top, Cin), jnp.bfloat16)
    scr_ref[top:top + M, :] = xrow
    scr_ref[top + M:, :] = jnp.zeros((scr_ref.shape[0] - top - M, Cin),
                                     jnp.bfloat16)
    xv = jnp.concatenate([scr_ref[o:o + M, :] for o in offs_v], axis=1)
    xh = jnp.concatenate([scr_ref[o:o + M, :] for o in offs_h], axis=1)
    yv = jnp.dot(xv, wv_ref[...], preferred_element_type=jnp.float32)
    yh = jnp.dot(xh, wh_ref[...], preferred_element_type=jnp.float32)
    ym = jnp.concatenate([yv, yh], axis=1)
    ys = ym.reshape(H, WG, -1)[:, _PAD:_PAD + W, :] + b_ref[...].reshape(1, 1, -1)
    feat_ref[0] = ys.reshape(H * W, -1).astype(feat_ref.dtype)
    ssum = jnp.sum(ys, axis=(0, 1)).reshape(1, -1)
    ssq = jnp.sum(ys * ys, axis=(0, 1)).reshape(1, -1)
    stats_ref[0] = jnp.concatenate([ssum, ssq], axis=0)


def _bn_relu_kernel(feat_ref, sc_ref, sh_ref, o_ref, *, H, W):
    z = jnp.maximum(feat_ref[0].astype(jnp.float32) * sc_ref[...] + sh_ref[...],
                    0.0)
    o_ref[0] = z.reshape(H, W, -1)


def _pack_taps(ws, dils, Cin):
    cg = sum(w.shape[0] for w in ws)
    wf = jnp.zeros((7 * Cin, cg), jnp.float32)
    co = 0
    for w, dil in zip(ws, dils):
        c = w.shape[0]
        w2 = w.reshape(c, Cin, _KSIZE).astype(jnp.float32)
        for t in range(_KSIZE):
            s = 3 + (t - 1) * dil
            wf = wf.at[s * Cin:(s + 1) * Cin, co:co + c].set(w2[:, :, t].T)
        co += c
    return wf


def kernel(x, w_first, w_second, w_third, w_first2, w_second2, w_third2,
           b_first, b_second, b_third, b_first2, b_second2, b_third2,
           gamma, beta):
    N, Cin, H, W = x.shape
    Cout = gamma.shape[0]

    WG = W + 2 * _PAD + 2
    HP = H + 2 * _PAD
    M = H * WG
    R = HP * WG

    xt = jnp.transpose(x, (0, 2, 3, 1))

    offs_v = tuple((_PAD + dh) * WG for dh in range(-3, 4))
    offs_h = tuple(_PAD * WG + dw for dw in range(-3, 4))

    wv = _pack_taps([w_first, w_second, w_third], (1, 2, 3), Cin).astype(jnp.bfloat16)
    wh = _pack_taps([w_first2, w_second2, w_third2], (1, 2, 3), Cin).astype(jnp.bfloat16)
    bias = jnp.concatenate([b_first, b_second, b_third,
                            b_first2, b_second2, b_third2])
    bias2 = bias.reshape(1, Cout).astype(jnp.float32)

    conv_fn = functools.partial(_conv_stats_kernel, offs_v=offs_v,
                                offs_h=offs_h, H=H, W=W, WG=WG)
    feat, stats = pl.pallas_call(
        conv_fn,
        out_shape=(jax.ShapeDtypeStruct((N, H * W, Cout), jnp.bfloat16),
                   jax.ShapeDtypeStruct((N, 2, Cout), jnp.float32)),
        grid=(N,),
        in_specs=[pl.BlockSpec((1, H, W, Cin), lambda n: (n, 0, 0, 0)),
                  pl.BlockSpec(wv.shape, lambda n: (0, 0)),
                  pl.BlockSpec(wh.shape, lambda n: (0, 0)),
                  pl.BlockSpec((1, Cout), lambda n: (0, 0))],
        out_specs=(pl.BlockSpec((1, H * W, Cout), lambda n: (n, 0, 0)),
                   pl.BlockSpec((1, 2, Cout), lambda n: (n, 0, 0))),
        scratch_shapes=[pltpu.VMEM((R, Cin), jnp.bfloat16)],
        compiler_params=pltpu.CompilerParams(dimension_semantics=("parallel",)),
    )(xt, wv, wh, bias2)

    tot = jnp.sum(stats, axis=0)
    inv_count = 1.0 / float(N * H * W)
    mean = tot[0] * inv_count
    var = tot[1] * inv_count - mean * mean
    scale = gamma.astype(jnp.float32) * jax.lax.rsqrt(var + jnp.float32(_BN_EPS))
    shift = beta.astype(jnp.float32) - mean * scale
    scale2 = scale.reshape(1, Cout)
    shift2 = shift.reshape(1, Cout)

    bn_fn = functools.partial(_bn_relu_kernel, H=H, W=W)
    out = pl.pallas_call(
        bn_fn,
        out_shape=jax.ShapeDtypeStruct((N, H, W, Cout), jnp.float32),
        grid=(N,),
        in_specs=[pl.BlockSpec((1, H * W, Cout), lambda n: (n, 0, 0)),
                  pl.BlockSpec((1, Cout), lambda n: (0, 0)),
                  pl.BlockSpec((1, Cout), lambda n: (0, 0))],
        out_specs=pl.BlockSpec((1, H, W, Cout), lambda n: (n, 0, 0, 0)),
        compiler_params=pltpu.CompilerParams(dimension_semantics=("parallel",)),
    )(feat, scale2, shift2)
    return jnp.transpose(out, (0, 3, 1, 2))


# speedup vs baseline: 3.0089x; 1.1128x over previous
import functools

import jax
import jax.numpy as jnp
from jax.experimental import pallas as pl
from jax.experimental.pallas import tpu as pltpu

_PAD = 3
_KSIZE = 3
_BN_EPS = 1e-5


def _fused_kernel(x_ref, wv_ref, wh_ref, b_ref, g_ref, bt_ref, o_ref,
                  scr_ref, feat_ref, stats_ref, *,
                  offs_v, offs_h, N, H, W, WG):
    M = H * WG
    Cin = x_ref.shape[3]
    j = pl.program_id(0)

    @pl.when(j == 0)
    def _init():
        stats_ref[...] = jnp.zeros_like(stats_ref)

    @pl.when(j < N)
    def _conv_phase():
        xr = x_ref[0].astype(jnp.bfloat16)
        zl = jnp.zeros((H, _PAD, Cin), jnp.bfloat16)
        zr = jnp.zeros((H, WG - W - _PAD, Cin), jnp.bfloat16)
        xrow = jnp.concatenate([zl, xr, zr], axis=1).reshape(M, Cin)
        top = _PAD * WG
        scr_ref[0:top, :] = jnp.zeros((top, Cin), jnp.bfloat16)
        scr_ref[top:top + M, :] = xrow
        scr_ref[top + M:, :] = jnp.zeros((scr_ref.shape[0] - top - M, Cin),
                                         jnp.bfloat16)
        xv = jnp.concatenate([scr_ref[o:o + M, :] for o in offs_v], axis=1)
        xh = jnp.concatenate([scr_ref[o:o + M, :] for o in offs_h], axis=1)
        yv = jnp.dot(xv, wv_ref[...], preferred_element_type=jnp.float32)
        yh = jnp.dot(xh, wh_ref[...], preferred_element_type=jnp.float32)
        ym = jnp.concatenate([yv, yh], axis=1)
        ys = (ym.reshape(H, WG, -1)[:, _PAD:_PAD + W, :]
              + b_ref[...].reshape(1, 1, -1))
        feat_ref[j] = ys.reshape(H * W, -1).astype(feat_ref.dtype)
        ssum = jnp.sum(ys, axis=(0, 1)).reshape(1, -1)
        ssq = jnp.sum(ys * ys, axis=(0, 1)).reshape(1, -1)
        stats_ref[...] += jnp.concatenate([ssum, ssq], axis=0)

    @pl.when(j >= N)
    def _bn_phase():
        inv_count = 1.0 / float(N * H * W)
        tot = stats_ref[...]
        mean = tot[0:1] * inv_count
        var = tot[1:2] * inv_count - mean * mean
        scale = g_ref[...] * jax.lax.rsqrt(var + jnp.float32(_BN_EPS))
        shift = bt_ref[...] - mean * scale
        fb = feat_ref[jnp.maximum(j - N, 0)].astype(jnp.float32)
        z = jnp.maximum(fb * scale + shift, 0.0)
        o_ref[0] = z.reshape(H, W, -1)


def _pack_taps(ws, dils, Cin):
    cg = sum(w.shape[0] for w in ws)
    wf = jnp.zeros((7 * Cin, cg), jnp.float32)
    co = 0
    for w, dil in zip(ws, dils):
        c = w.shape[0]
        w2 = w.reshape(c, Cin, _KSIZE).astype(jnp.float32)
        for t in range(_KSIZE):
            s = 3 + (t - 1) * dil
            wf = wf.at[s * Cin:(s + 1) * Cin, co:co + c].set(w2[:, :, t].T)
        co += c
    return wf


def kernel(x, w_first, w_second, w_third, w_first2, w_second2, w_third2,
           b_first, b_second, b_third, b_first2, b_second2, b_third2,
           gamma, beta):
    N, Cin, H, W = x.shape
    Cout = gamma.shape[0]

    WG = W + 2 * _PAD + 2
    HP = H + 2 * _PAD
    M = H * WG
    R = HP * WG

    xt = jnp.transpose(x, (0, 2, 3, 1))

    offs_v = tuple((_PAD + dh) * WG for dh in range(-3, 4))
    offs_h = tuple(_PAD * WG + dw for dw in range(-3, 4))

    wv = _pack_taps([w_first, w_second, w_third], (1, 2, 3), Cin).astype(jnp.bfloat16)
    wh = _pack_taps([w_first2, w_second2, w_third2], (1, 2, 3), Cin).astype(jnp.bfloat16)
    bias = jnp.concatenate([b_first, b_second, b_third,
                            b_first2, b_second2, b_third2])
    bias2 = bias.reshape(1, Cout).astype(jnp.float32)
    gamma2 = gamma.reshape(1, Cout).astype(jnp.float32)
    beta2 = beta.reshape(1, Cout).astype(jnp.float32)

    fused_fn = functools.partial(_fused_kernel, offs_v=offs_v, offs_h=offs_h,
                                 N=N, H=H, W=W, WG=WG)
    out = pl.pallas_call(
        fused_fn,
        out_shape=jax.ShapeDtypeStruct((N, H, W, Cout), jnp.float32),
        grid=(2 * N,),
        in_specs=[pl.BlockSpec((1, H, W, Cin),
                               lambda j: (jnp.minimum(j, N - 1), 0, 0, 0)),
                  pl.BlockSpec(wv.shape, lambda j: (0, 0)),
                  pl.BlockSpec(wh.shape, lambda j: (0, 0)),
                  pl.BlockSpec((1, Cout), lambda j: (0, 0)),
                  pl.BlockSpec((1, Cout), lambda j: (0, 0)),
                  pl.BlockSpec((1, Cout), lambda j: (0, 0))],
        out_specs=pl.BlockSpec((1, H, W, Cout),
                               lambda j: (jnp.maximum(j - N, 0), 0, 0, 0)),
        scratch_shapes=[pltpu.VMEM((R, Cin), jnp.bfloat16),
                        pltpu.VMEM((N, H * W, Cout), jnp.bfloat16),
                        pltpu.VMEM((2, Cout), jnp.float32)],
        compiler_params=pltpu.CompilerParams(
            dimension_semantics=("arbitrary",)),
    )(xt, wv, wh, bias2, gamma2, beta2)
    return jnp.transpose(out, (0, 3, 1, 2))


# speedup vs baseline: 3.0136x; 1.0016x over previous
import functools

import jax
import jax.numpy as jnp
from jax.experimental import pallas as pl
from jax.experimental.pallas import tpu as pltpu

_PAD = 3
_KSIZE = 3
_BN_EPS = 1e-5


def _fused_kernel(x_ref, wv_ref, wh_ref, b_ref, g_ref, bt_ref, o_ref,
                  scr_ref, feat_ref, stats_ref, *,
                  offs_v, offs_h, N, H, W, WG):
    M = H * WG
    Cin = x_ref.shape[3]
    j = pl.program_id(0)

    top = _PAD * WG

    @pl.when(j == 0)
    def _init():
        stats_ref[...] = jnp.zeros_like(stats_ref)
        scr_ref[0:top, :] = jnp.zeros((top, Cin), jnp.bfloat16)
        scr_ref[top + M:, :] = jnp.zeros((scr_ref.shape[0] - top - M, Cin),
                                         jnp.bfloat16)

    @pl.when(j < N)
    def _conv_phase():
        xr = x_ref[0].astype(jnp.bfloat16)
        zl = jnp.zeros((H, _PAD, Cin), jnp.bfloat16)
        zr = jnp.zeros((H, WG - W - _PAD, Cin), jnp.bfloat16)
        xrow = jnp.concatenate([zl, xr, zr], axis=1).reshape(M, Cin)
        scr_ref[top:top + M, :] = xrow
        xv = jnp.concatenate([scr_ref[o:o + M, :] for o in offs_v], axis=1)
        xh = jnp.concatenate([scr_ref[o:o + M, :] for o in offs_h], axis=1)
        yv = jnp.dot(xv, wv_ref[...], preferred_element_type=jnp.float32)
        yh = jnp.dot(xh, wh_ref[...], preferred_element_type=jnp.float32)
        ym = jnp.concatenate([yv, yh], axis=1)
        ys = (ym.reshape(H, WG, -1)[:, _PAD:_PAD + W, :]
              + b_ref[...].reshape(1, 1, -1))
        feat_ref[j] = ys.reshape(H * W, -1).astype(feat_ref.dtype)
        ssum = jnp.sum(ys, axis=(0, 1)).reshape(1, -1)
        ssq = jnp.sum(ys * ys, axis=(0, 1)).reshape(1, -1)
        stats_ref[...] += jnp.concatenate([ssum, ssq], axis=0)

    @pl.when(j >= N)
    def _bn_phase():
        inv_count = 1.0 / float(N * H * W)
        tot = stats_ref[...]
        mean = tot[0:1] * inv_count
        var = tot[1:2] * inv_count - mean * mean
        scale = g_ref[...] * jax.lax.rsqrt(var + jnp.float32(_BN_EPS))
        shift = bt_ref[...] - mean * scale
        fb = feat_ref[jnp.maximum(j - N, 0)].astype(jnp.float32)
        z = jnp.maximum(fb * scale + shift, 0.0)
        o_ref[0] = z.reshape(H, W, -1)


def _pack_taps(ws, dils, Cin):
    cg = sum(w.shape[0] for w in ws)
    wf = jnp.zeros((7 * Cin, cg), jnp.float32)
    co = 0
    for w, dil in zip(ws, dils):
        c = w.shape[0]
        w2 = w.reshape(c, Cin, _KSIZE).astype(jnp.float32)
        for t in range(_KSIZE):
            s = 3 + (t - 1) * dil
            wf = wf.at[s * Cin:(s + 1) * Cin, co:co + c].set(w2[:, :, t].T)
        co += c
    return wf


def kernel(x, w_first, w_second, w_third, w_first2, w_second2, w_third2,
           b_first, b_second, b_third, b_first2, b_second2, b_third2,
           gamma, beta):
    N, Cin, H, W = x.shape
    Cout = gamma.shape[0]

    WG = W + 2 * _PAD + 2
    HP = H + 2 * _PAD
    M = H * WG
    R = HP * WG

    xt = jnp.transpose(x, (0, 2, 3, 1))

    offs_v = tuple((_PAD + dh) * WG for dh in range(-3, 4))
    offs_h = tuple(_PAD * WG + dw for dw in range(-3, 4))

    wv = _pack_taps([w_first, w_second, w_third], (1, 2, 3), Cin).astype(jnp.bfloat16)
    wh = _pack_taps([w_first2, w_second2, w_third2], (1, 2, 3), Cin).astype(jnp.bfloat16)
    bias = jnp.concatenate([b_first, b_second, b_third,
                            b_first2, b_second2, b_third2])
    bias2 = bias.reshape(1, Cout).astype(jnp.float32)
    gamma2 = gamma.reshape(1, Cout).astype(jnp.float32)
    beta2 = beta.reshape(1, Cout).astype(jnp.float32)

    fused_fn = functools.partial(_fused_kernel, offs_v=offs_v, offs_h=offs_h,
                                 N=N, H=H, W=W, WG=WG)
    out = pl.pallas_call(
        fused_fn,
        out_shape=jax.ShapeDtypeStruct((N, H, W, Cout), jnp.float32),
        grid=(2 * N,),
        in_specs=[pl.BlockSpec((1, H, W, Cin),
                               lambda j: (jnp.minimum(j, N - 1), 0, 0, 0)),
                  pl.BlockSpec(wv.shape, lambda j: (0, 0)),
                  pl.BlockSpec(wh.shape, lambda j: (0, 0)),
                  pl.BlockSpec((1, Cout), lambda j: (0, 0)),
                  pl.BlockSpec((1, Cout), lambda j: (0, 0)),
                  pl.BlockSpec((1, Cout), lambda j: (0, 0))],
        out_specs=pl.BlockSpec((1, H, W, Cout),
                               lambda j: (jnp.maximum(j - N, 0), 0, 0, 0)),
        scratch_shapes=[pltpu.VMEM((R, Cin), jnp.bfloat16),
                        pltpu.VMEM((N, H * W, Cout), jnp.bfloat16),
                        pltpu.VMEM((2, Cout), jnp.float32)],
        compiler_params=pltpu.CompilerParams(
            dimension_semantics=("arbitrary",)),
    )(xt, wv, wh, bias2, gamma2, beta2)
    return jnp.transpose(out, (0, 3, 1, 2))


# speedup vs baseline: 3.3918x; 1.1255x over previous
import functools

import jax
import jax.numpy as jnp
from jax.experimental import pallas as pl
from jax.experimental.pallas import tpu as pltpu

_PAD = 3
_KSIZE = 3
_BN_EPS = 1e-5


def _fused_kernel(x_ref, wv_ref, wh_ref, b_ref, g_ref, bt_ref, o_ref,
                  scr_ref, feat_ref, stats_ref, *,
                  offs_v, offs_h, N, H, W, WG):
    M = H * WG
    Cin = x_ref.shape[3]
    j = pl.program_id(0)

    top = _PAD * WG

    @pl.when(j == 0)
    def _init():
        stats_ref[...] = jnp.zeros_like(stats_ref)
        scr_ref[0:top, :] = jnp.zeros((top, Cin), jnp.bfloat16)
        scr_ref[top + M:, :] = jnp.zeros((scr_ref.shape[0] - top - M, Cin),
                                         jnp.bfloat16)

    @pl.when(j < N)
    def _conv_phase():
        xr = x_ref[0].astype(jnp.bfloat16)
        zl = jnp.zeros((H, _PAD, Cin), jnp.bfloat16)
        zr = jnp.zeros((H, WG - W - _PAD, Cin), jnp.bfloat16)
        xrow = jnp.concatenate([zl, xr, zr], axis=1).reshape(M, Cin)
        scr_ref[top:top + M, :] = xrow
        xv = jnp.concatenate([scr_ref[o:o + M, :] for o in offs_v], axis=1)
        xh = jnp.concatenate([scr_ref[o:o + M, :] for o in offs_h], axis=1)
        yv = jnp.dot(xv, wv_ref[...], preferred_element_type=jnp.float32)
        yh = jnp.dot(xh, wh_ref[...], preferred_element_type=jnp.float32)
        ym = jnp.concatenate([yv, yh], axis=1)
        ys = (ym.reshape(H, WG, -1)[:, _PAD:_PAD + W, :]
              + b_ref[...].reshape(1, 1, -1))
        feat_ref[j] = ys.reshape(H * W, -1).astype(feat_ref.dtype)
        ssum = jnp.sum(ys, axis=(0, 1)).reshape(1, -1)
        ssq = jnp.sum(ys * ys, axis=(0, 1)).reshape(1, -1)
        stats_ref[...] += jnp.concatenate([ssum, ssq], axis=0)

    @pl.when(j >= N)
    def _bn_phase():
        inv_count = 1.0 / float(N * H * W)
        tot = stats_ref[...]
        mean = tot[0:1] * inv_count
        var = tot[1:2] * inv_count - mean * mean
        scale = g_ref[...] * jax.lax.rsqrt(var + jnp.float32(_BN_EPS))
        shift = bt_ref[...] - mean * scale
        fb = feat_ref[jnp.maximum(j - N, 0)].astype(jnp.float32)
        z = jnp.maximum(fb * scale + shift, 0.0)
        o_ref[0] = z.reshape(H, W, -1)


def _pack_taps(ws, dils, Cin):
    cols = []
    for w, dil in zip(ws, dils):
        c = w.shape[0]
        w2 = w.reshape(c, Cin, _KSIZE).astype(jnp.float32)
        zero = jnp.zeros((Cin, c), jnp.float32)
        slots = {3 + (t - 1) * dil: w2[:, :, t].T for t in range(_KSIZE)}
        cols.append(jnp.concatenate([slots.get(s, zero) for s in range(7)],
                                    axis=0))
    return jnp.concatenate(cols, axis=1)


def kernel(x, w_first, w_second, w_third, w_first2, w_second2, w_third2,
           b_first, b_second, b_third, b_first2, b_second2, b_third2,
           gamma, beta):
    N, Cin, H, W = x.shape
    Cout = gamma.shape[0]

    WG = W + 2 * _PAD + 2
    HP = H + 2 * _PAD
    M = H * WG
    R = HP * WG

    xt = jnp.transpose(x, (0, 2, 3, 1))

    offs_v = tuple((_PAD + dh) * WG for dh in range(-3, 4))
    offs_h = tuple(_PAD * WG + dw for dw in range(-3, 4))

    wv = _pack_taps([w_first, w_second, w_third], (1, 2, 3), Cin).astype(jnp.bfloat16)
    wh = _pack_taps([w_first2, w_second2, w_third2], (1, 2, 3), Cin).astype(jnp.bfloat16)
    bias = jnp.concatenate([b_first, b_second, b_third,
                            b_first2, b_second2, b_third2])
    bias2 = bias.reshape(1, Cout).astype(jnp.float32)
    gamma2 = gamma.reshape(1, Cout).astype(jnp.float32)
    beta2 = beta.reshape(1, Cout).astype(jnp.float32)

    fused_fn = functools.partial(_fused_kernel, offs_v=offs_v, offs_h=offs_h,
                                 N=N, H=H, W=W, WG=WG)
    out = pl.pallas_call(
        fused_fn,
        out_shape=jax.ShapeDtypeStruct((N, H, W, Cout), jnp.float32),
        grid=(2 * N,),
        in_specs=[pl.BlockSpec((1, H, W, Cin),
                               lambda j: (jnp.minimum(j, N - 1), 0, 0, 0)),
                  pl.BlockSpec(wv.shape, lambda j: (0, 0)),
                  pl.BlockSpec(wh.shape, lambda j: (0, 0)),
                  pl.BlockSpec((1, Cout), lambda j: (0, 0)),
                  pl.BlockSpec((1, Cout), lambda j: (0, 0)),
                  pl.BlockSpec((1, Cout), lambda j: (0, 0))],
        out_specs=pl.BlockSpec((1, H, W, Cout),
                               lambda j: (jnp.maximum(j - N, 0), 0, 0, 0)),
        scratch_shapes=[pltpu.VMEM((R, Cin), jnp.bfloat16),
                        pltpu.VMEM((N, H * W, Cout), jnp.bfloat16),
                        pltpu.VMEM((2, Cout), jnp.float32)],
        compiler_params=pltpu.CompilerParams(
            dimension_semantics=("arbitrary",)),
    )(xt, wv, wh, bias2, gamma2, beta2)
    return jnp.transpose(out, (0, 3, 1, 2))


# speedup vs baseline: 3.5303x; 1.0408x over previous
import functools

import jax
import jax.numpy as jnp
from jax.experimental import pallas as pl
from jax.experimental.pallas import tpu as pltpu

_PAD = 3
_KSIZE = 3
_BN_EPS = 1e-5


def _fused_kernel(x_ref, wv_ref, wh_ref, b_ref, g_ref, bt_ref, o_ref,
                  scr_ref, feat_ref, stats_ref, *,
                  offs_v, offs_h, N, H, W, WG):
    M = H * WG
    Cin = x_ref.shape[3]
    j = pl.program_id(0)

    top = _PAD * WG

    @pl.when(j == 0)
    def _init():
        stats_ref[...] = jnp.zeros_like(stats_ref)
        scr_ref[0:top, :] = jnp.zeros((top, Cin), jnp.bfloat16)
        scr_ref[top + M:, :] = jnp.zeros((scr_ref.shape[0] - top - M, Cin),
                                         jnp.bfloat16)

    @pl.when(j < N)
    def _conv_phase():
        xr = x_ref[0].astype(jnp.bfloat16)
        zl = jnp.zeros((H, _PAD, Cin), jnp.bfloat16)
        zr = jnp.zeros((H, WG - W - _PAD, Cin), jnp.bfloat16)
        xrow = jnp.concatenate([zl, xr, zr], axis=1).reshape(M, Cin)
        scr_ref[top:top + M, :] = xrow
        xv = jnp.concatenate([scr_ref[o:o + M, :] for o in offs_v], axis=1)
        xh = jnp.concatenate([scr_ref[o:o + M, :] for o in offs_h], axis=1)
        yv = jnp.dot(xv, wv_ref[...], preferred_element_type=jnp.float32)
        yh = jnp.dot(xh, wh_ref[...], preferred_element_type=jnp.float32)
        ym = jnp.concatenate([yv, yh], axis=1)
        ys = (ym.reshape(H, WG, -1)[:, _PAD:_PAD + W, :]
              + b_ref[...].reshape(1, 1, -1))
        feat_ref[j] = ys.reshape(H * W, -1).astype(feat_ref.dtype)
        ssum = jnp.sum(ys, axis=(0, 1)).reshape(1, -1)
        ssq = jnp.sum(ys * ys, axis=(0, 1)).reshape(1, -1)
        stats_ref[...] += jnp.concatenate([ssum, ssq], axis=0)

    @pl.when(j >= N)
    def _bn_phase():
        B = o_ref.shape[0]
        inv_count = 1.0 / float(N * H * W)
        tot = stats_ref[...]
        mean = tot[0:1] * inv_count
        var = tot[1:2] * inv_count - mean * mean
        scale = g_ref[...] * jax.lax.rsqrt(var + jnp.float32(_BN_EPS))
        shift = bt_ref[...] - mean * scale
        fb = feat_ref[pl.ds(jnp.maximum(j - N, 0) * B, B)].astype(jnp.float32)
        z = jnp.maximum(fb * scale.reshape(1, 1, -1) + shift.reshape(1, 1, -1),
                        0.0)
        o_ref[...] = z.reshape(B, H, W, -1)


def _pack_taps(ws, dils, Cin):
    cols = []
    for w, dil in zip(ws, dils):
        c = w.shape[0]
        w2 = w.reshape(c, Cin, _KSIZE).astype(jnp.float32)
        zero = jnp.zeros((Cin, c), jnp.float32)
        slots = {3 + (t - 1) * dil: w2[:, :, t].T for t in range(_KSIZE)}
        cols.append(jnp.concatenate([slots.get(s, zero) for s in range(7)],
                                    axis=0))
    return jnp.concatenate(cols, axis=1)


def kernel(x, w_first, w_second, w_third, w_first2, w_second2, w_third2,
           b_first, b_second, b_third, b_first2, b_second2, b_third2,
           gamma, beta):
    N, Cin, H, W = x.shape
    Cout = gamma.shape[0]

    WG = W + 2 * _PAD + 2
    HP = H + 2 * _PAD
    M = H * WG
    R = HP * WG

    xt = jnp.transpose(x, (0, 2, 3, 1))

    offs_v = tuple((_PAD + dh) * WG for dh in range(-3, 4))
    offs_h = tuple(_PAD * WG + dw for dw in range(-3, 4))

    wv = _pack_taps([w_first, w_second, w_third], (1, 2, 3), Cin).astype(jnp.bfloat16)
    wh = _pack_taps([w_first2, w_second2, w_third2], (1, 2, 3), Cin).astype(jnp.bfloat16)
    bias = jnp.concatenate([b_first, b_second, b_third,
                            b_first2, b_second2, b_third2])
    bias2 = bias.reshape(1, Cout).astype(jnp.float32)
    gamma2 = gamma.reshape(1, Cout).astype(jnp.float32)
    beta2 = beta.reshape(1, Cout).astype(jnp.float32)

    fused_fn = functools.partial(_fused_kernel, offs_v=offs_v, offs_h=offs_h,
                                 N=N, H=H, W=W, WG=WG)
    B = 2 if N % 2 == 0 else 1
    out = pl.pallas_call(
        fused_fn,
        out_shape=jax.ShapeDtypeStruct((N, H, W, Cout), jnp.float32),
        grid=(N + N // B,),
        in_specs=[pl.BlockSpec((1, H, W, Cin),
                               lambda j: (jnp.minimum(j, N - 1), 0, 0, 0)),
                  pl.BlockSpec(wv.shape, lambda j: (0, 0)),
                  pl.BlockSpec(wh.shape, lambda j: (0, 0)),
                  pl.BlockSpec((1, Cout), lambda j: (0, 0)),
                  pl.BlockSpec((1, Cout), lambda j: (0, 0)),
                  pl.BlockSpec((1, Cout), lambda j: (0, 0))],
        out_specs=pl.BlockSpec((B, H, W, Cout),
                               lambda j: (jnp.maximum(j - N, 0), 0, 0, 0)),
        scratch_shapes=[pltpu.VMEM((R, Cin), jnp.bfloat16),
                        pltpu.VMEM((N, H * W, Cout), jnp.bfloat16),
                        pltpu.VMEM((2, Cout), jnp.float32)],
        compiler_params=pltpu.CompilerParams(
            dimension_semantics=("arbitrary",)),
    )(xt, wv, wh, bias2, gamma2, beta2)
    return jnp.transpose(out, (0, 3, 1, 2))


# speedup vs baseline: 3.5614x; 1.0088x over previous
import functools

import jax
import jax.numpy as jnp
from jax.experimental import pallas as pl
from jax.experimental.pallas import tpu as pltpu

_PAD = 3
_KSIZE = 3
_BN_EPS = 1e-5


def _fused_kernel(x_ref, wv_ref, wh_ref, b_ref, g_ref, bt_ref, o_ref,
                  scr_ref, feat_ref, stats_ref, *,
                  offs_v, offs_h, N, H, W, WG):
    M = H * WG
    Cin = x_ref.shape[3]
    j = pl.program_id(0)

    top = _PAD * WG

    @pl.when(j == 0)
    def _init():
        stats_ref[...] = jnp.zeros_like(stats_ref)
        scr_ref[0:top, :] = jnp.zeros((top, Cin), jnp.bfloat16)
        scr_ref[top + M:, :] = jnp.zeros((scr_ref.shape[0] - top - M, Cin),
                                         jnp.bfloat16)

    @pl.when(j < N)
    def _conv_phase():
        xr = x_ref[0].astype(jnp.bfloat16)
        zl = jnp.zeros((H, _PAD, Cin), jnp.bfloat16)
        zr = jnp.zeros((H, WG - W - _PAD, Cin), jnp.bfloat16)
        xrow = jnp.concatenate([zl, xr, zr], axis=1).reshape(M, Cin)
        scr_ref[top:top + M, :] = xrow
        xv = jnp.concatenate([scr_ref[o:o + M, :] for o in offs_v], axis=1)
        xh = jnp.concatenate([scr_ref[o:o + M, :] for o in offs_h], axis=1)
        yv = jnp.dot(xv, wv_ref[...], preferred_element_type=jnp.float32)
        yh = jnp.dot(xh, wh_ref[...], preferred_element_type=jnp.float32)
        ym = jnp.concatenate([yv, yh], axis=1)
        ys = (ym.reshape(H, WG, -1)[:, _PAD:_PAD + W, :]
              + b_ref[...].reshape(1, 1, -1))
        feat_ref[j] = ys.reshape(H * W, -1).astype(feat_ref.dtype)
        ssum = jnp.sum(ys, axis=(0, 1)).reshape(1, -1)
        ssq = jnp.sum(ys * ys, axis=(0, 1)).reshape(1, -1)
        stats_ref[...] += jnp.concatenate([ssum, ssq], axis=0)

    @pl.when(j >= N)
    def _bn_phase():
        B = o_ref.shape[0]
        inv_count = 1.0 / float(N * H * W)
        tot = stats_ref[...]
        mean = tot[0:1] * inv_count
        var = tot[1:2] * inv_count - mean * mean
        scale = g_ref[...] * jax.lax.rsqrt(var + jnp.float32(_BN_EPS))
        shift = bt_ref[...] - mean * scale
        fb = feat_ref[pl.ds(jnp.maximum(j - N, 0) * B, B)].astype(jnp.float32)
        z = jnp.maximum(fb * scale.reshape(1, 1, -1) + shift.reshape(1, 1, -1),
                        0.0)
        o_ref[...] = z.reshape(B, H, W, -1)


def _pack_taps(ws, dils, Cin):
    cols = []
    for w, dil in zip(ws, dils):
        c = w.shape[0]
        w2 = w.reshape(c, Cin, _KSIZE).astype(jnp.float32)
        zero = jnp.zeros((Cin, c), jnp.float32)
        slots = {3 + (t - 1) * dil: w2[:, :, t].T for t in range(_KSIZE)}
        cols.append(jnp.concatenate([slots.get(s, zero) for s in range(7)],
                                    axis=0))
    return jnp.concatenate(cols, axis=1)


def kernel(x, w_first, w_second, w_third, w_first2, w_second2, w_third2,
           b_first, b_second, b_third, b_first2, b_second2, b_third2,
           gamma, beta):
    N, Cin, H, W = x.shape
    Cout = gamma.shape[0]

    WG = W + 2 * _PAD + 2
    HP = H + 2 * _PAD
    M = H * WG
    R = HP * WG

    xt = jnp.transpose(x, (0, 2, 3, 1))

    offs_v = tuple((_PAD + dh) * WG for dh in range(-3, 4))
    offs_h = tuple(_PAD * WG + dw for dw in range(-3, 4))

    wv = _pack_taps([w_first, w_second, w_third], (1, 2, 3), Cin).astype(jnp.bfloat16)
    wh = _pack_taps([w_first2, w_second2, w_third2], (1, 2, 3), Cin).astype(jnp.bfloat16)
    bias = jnp.concatenate([b_first, b_second, b_third,
                            b_first2, b_second2, b_third2])
    bias2 = bias.reshape(1, Cout).astype(jnp.float32)
    gamma2 = gamma.reshape(1, Cout).astype(jnp.float32)
    beta2 = beta.reshape(1, Cout).astype(jnp.float32)

    fused_fn = functools.partial(_fused_kernel, offs_v=offs_v, offs_h=offs_h,
                                 N=N, H=H, W=W, WG=WG)
    B = next(b for b in (4, 2, 1) if N % b == 0)
    out = pl.pallas_call(
        fused_fn,
        out_shape=jax.ShapeDtypeStruct((N, H, W, Cout), jnp.float32),
        grid=(N + N // B,),
        in_specs=[pl.BlockSpec((1, H, W, Cin),
                               lambda j: (jnp.minimum(j, N - 1), 0, 0, 0)),
                  pl.BlockSpec(wv.shape, lambda j: (0, 0)),
                  pl.BlockSpec(wh.shape, lambda j: (0, 0)),
                  pl.BlockSpec((1, Cout), lambda j: (0, 0)),
                  pl.BlockSpec((1, Cout), lambda j: (0, 0)),
                  pl.BlockSpec((1, Cout), lambda j: (0, 0))],
        out_specs=pl.BlockSpec((B, H, W, Cout),
                               lambda j: (jnp.maximum(j - N, 0), 0, 0, 0)),
        scratch_shapes=[pltpu.VMEM((R, Cin), jnp.bfloat16),
                        pltpu.VMEM((N, H * W, Cout), jnp.bfloat16),
                        pltpu.VMEM((2, Cout), jnp.float32)],
        compiler_params=pltpu.CompilerParams(
            dimension_semantics=("arbitrary",)),
    )(xt, wv, wh, bias2, gamma2, beta2)
    return jnp.transpose(out, (0, 3, 1, 2))
